# baseline (speedup 1.0000x reference)
import jax
import jax.numpy as jnp
from jax.experimental import pallas as pl


def _leaky_relu(x, slope=0.2):
    return jnp.where(x >= 0, x, slope * x)


def _gat_conv(x, edge_index, W, a_src, a_dst, b):
    N = x.shape[0]
    loops = jnp.arange(N, dtype=edge_index.dtype)
    src = jnp.concatenate([edge_index[0], loops])
    dst = jnp.concatenate([edge_index[1], loops])
    h = x @ W.T
    alpha_s = h @ a_src
    alpha_d = h @ a_dst
    e = _leaky_relu(alpha_s[src] + alpha_d[dst], 0.2)
    m = jax.ops.segment_max(e, dst, num_segments=N)
    ex = jnp.exp(e - m[dst])
    denom = jax.ops.segment_sum(ex, dst, num_segments=N)
    coef = ex / (denom[dst] + 1e-16)
    out = jax.ops.segment_sum(h[src] * coef[:, None], dst, num_segments=N)
    return out + b


def _linear_kernel(h_ref, w_ref, b_ref, o_ref):
    o_ref[...] = h_ref[...] @ w_ref[...] + b_ref[...]


def kernel(x, edge_index, batch, W1, a1_src, a1_dst, b1, W2, a2_src, a2_dst, b2, Wl, bl):
    h = _gat_conv(x, edge_index, W1, a1_src, a1_dst, b1)
    h = jax.nn.relu(h)
    h = _gat_conv(h, edge_index, W2, a2_src, a2_dst, b2)
    N = h.shape[0]
    out = pl.pallas_call(
        _linear_kernel,
        out_shape=jax.ShapeDtypeStruct((N, Wl.shape[0]), jnp.float32),
        grid=(N // 10000,),
        in_specs=[
            pl.BlockSpec((10000, 64), lambda i: (i, 0)),
            pl.BlockSpec((64, 40), lambda i: (0, 0)),
            pl.BlockSpec((1, 40), lambda i: (0, 0)),
        ],
        out_specs=pl.BlockSpec((10000, 40), lambda i: (i, 0)),
    )(h, Wl.T, bl[None, :])
    return out


# speedup vs baseline: 42.1872x; 42.1872x over previous
import functools

import jax
import jax.numpy as jnp
from jax import lax
from jax.experimental import pallas as pl
from jax.experimental.pallas import tpu as pltpu
from jax.experimental.pallas import tpu_sc as plsc

N_NODES = 100000
NP = 100096
ROWS_TILE = 6256
TC_BLOCK = 2944
LANE = 128
RPS = 4
STEPS = 202
NTILES = 32
E_PAD = NTILES * STEPS * RPS * LANE
R_EDGE = E_PAD // LANE
ROWS_PER_TILE = STEPS * RPS


def _dense1_body(x_ref, w_ref, a_ref, h_ref, aa_ref):
    h = x_ref[...] @ w_ref[...]
    h_ref[...] = h
    aa_ref[...] = h @ a_ref[...]


def _dense1(x, W1, a1_src, a1_dst):
    N = x.shape[0]
    B = 10000
    h1, aa1 = pl.pallas_call(
        _dense1_body,
        out_shape=(
            jax.ShapeDtypeStruct((N, 16), jnp.float32),
            jax.ShapeDtypeStruct((N, 2), jnp.float32),
        ),
        grid=(N // B,),
        in_specs=[
            pl.BlockSpec((B, 11), lambda i: (i, 0)),
            pl.BlockSpec((11, 16), lambda i: (0, 0)),
            pl.BlockSpec((16, 2), lambda i: (0, 0)),
        ],
        out_specs=(
            pl.BlockSpec((B, 16), lambda i: (i, 0)),
            pl.BlockSpec((B, 2), lambda i: (i, 0)),
        ),
    )(x, W1.T, jnp.stack([a1_src, a1_dst], axis=1))
    return h1, aa1


def _combine1_body(num_ref, den_ref, b_ref, w_ref, a_ref, hc_ref, aa_ref):
    n = num_ref[0] + num_ref[1]
    d = den_ref[0] + den_ref[1]
    h = n / (d[:, None] + 1e-16) + b_ref[...]
    h = jnp.maximum(h, 0.0)
    h2 = h @ w_ref[...]
    for c in range(4):
        hc_ref[c] = h2[:, c * 16:(c + 1) * 16]
    aa_ref[...] = h2 @ a_ref[...]


def _combine1(numer1, denom1, b1, W2, a2_src, a2_dst):
    B = TC_BLOCK
    hc, aa2 = pl.pallas_call(
        _combine1_body,
        out_shape=(
            jax.ShapeDtypeStruct((4, NP, 16), jnp.float32),
            jax.ShapeDtypeStruct((NP, 2), jnp.float32),
        ),
        grid=(NP // B,),
        in_specs=[
            pl.BlockSpec((2, B, 16), lambda i: (0, i, 0)),
            pl.BlockSpec((2, B), lambda i: (0, i)),
            pl.BlockSpec((1, 16), lambda i: (0, 0)),
            pl.BlockSpec((16, 64), lambda i: (0, 0)),
            pl.BlockSpec((64, 2), lambda i: (0, 0)),
        ],
        out_specs=(
            pl.BlockSpec((4, B, 16), lambda i: (0, i, 0)),
            pl.BlockSpec((B, 2), lambda i: (i, 0)),
        ),
    )(numer1, denom1, b1[None, :], W2.T, jnp.stack([a2_src, a2_dst], axis=1))
    return hc, aa2


def _combine2_body(num_ref, den_ref, b_ref, w_ref, bl_ref, o_ref):
    n = num_ref[0] + num_ref[1]
    d = den_ref[0] + den_ref[1]
    h2 = jnp.concatenate([n[0], n[1], n[2], n[3]], axis=1)
    h = h2 / (d[:, None] + 1e-16) + b_ref[...]
    o_ref[...] = h @ w_ref[...] + bl_ref[...]


def _combine2(numer2, denom2, b2, Wl, bl):
    B = TC_BLOCK
    out = pl.pallas_call(
        _combine2_body,
        out_shape=jax.ShapeDtypeStruct((NP, 40), jnp.float32),
        grid=(NP // B,),
        in_specs=[
            pl.BlockSpec((2, 4, B, 16), lambda i: (0, 0, i, 0)),
            pl.BlockSpec((2, B), lambda i: (0, i)),
            pl.BlockSpec((1, 64), lambda i: (0, 0)),
            pl.BlockSpec((64, 40), lambda i: (0, 0)),
            pl.BlockSpec((1, 40), lambda i: (0, 0)),
        ],
        out_specs=pl.BlockSpec((B, 40), lambda i: (i, 0)),
    )(numer2, denom2, b2[None, :], Wl.T, bl[None, :])
    return out


def _sc_edge_body(C, E_prime, *refs):
    (src_hbm, dst_hbm) = refs[0:2]
    hs = refs[2:2 + C]
    as_hbm, ad_hbm, zrows_hbm, zden_hbm = refs[2 + C:6 + C]
    numer_hbm, denom_hbm, w_hbm = refs[6 + C:9 + C]
    (accum_sp, denom_sp, src_v, dst_v, wv, asg, adg, hrows,
     zv, vtmp, vtmpd, sem_a, sem_h) = refs[9 + C:]

    c_id = lax.axis_index("c")
    s_id = lax.axis_index("s")
    wid = c_id * 16 + s_id
    tile_row0 = wid * ROWS_PER_TILE
    out_r0 = s_id * ROWS_TILE
    BCH = 136
    NCH = ROWS_TILE // BCH

    pltpu.sync_copy(zrows_hbm, zv)
    for k in range(NCH):
        pltpu.sync_copy(zv, accum_sp.at[pl.ds(out_r0 + k * BCH, BCH)])
    pltpu.sync_copy(zden_hbm, vtmpd)
    pltpu.sync_copy(vtmpd, denom_sp.at[pl.ds(out_r0, ROWS_TILE)])

    for c in range(C):
        plsc.subcore_barrier()

        def step_fn(step, carry, c=c):
            row0 = tile_row0 + step * RPS
            pltpu.sync_copy(src_hbm.at[pl.ds(row0, RPS)], src_v)
            pltpu.sync_copy(dst_hbm.at[pl.ds(row0, RPS)], dst_v)

            hdescs = [pltpu.async_copy(hs[c].at[src_v.at[j]],
                                       hrows.at[j], sem_h)
                      for j in range(RPS)]

            if c == 0:
                descs = []
                for j in range(RPS):
                    descs.append(pltpu.async_copy(
                        as_hbm.at[src_v.at[j]], asg.at[j], sem_a))
                    descs.append(pltpu.async_copy(
                        ad_hbm.at[dst_v.at[j]], adg.at[j], sem_a))
                for dsc in descs:
                    dsc.wait()
                for j in range(RPS):
                    for r in range(LANE // 16):
                        sl = pl.ds(r * 16, 16)
                        e = asg[j, sl] + adg[j, sl]
                        e = jnp.where(e >= 0.0, e, 0.2 * e)
                        w = jnp.exp(e)
                        gbase = (row0 + j) * LANE + r * 16
                        ids = gbase + lax.iota(jnp.int32, 16)
                        wv[j, sl] = jnp.where(ids < E_prime, w, 0.0)
                pltpu.sync_copy(wv, w_hbm.at[pl.ds(row0, RPS)])
                for j in range(RPS):
                    pltpu.sync_copy(wv.at[j], denom_sp.at[dst_v.at[j]],
                                    add=True)
            else:
                pltpu.sync_copy(w_hbm.at[pl.ds(row0, RPS)], wv)

            for dsc in hdescs:
                dsc.wait()

            for j in range(RPS):
                for r16 in range(LANE // 16):
                    w16 = wv[j, pl.ds(r16 * 16, 16)]
                    for r in range(16):
                        row = r16 * 16 + r
                        b = jnp.take_along_axis(
                            w16, jnp.full((16,), r, jnp.int32), axis=0)
                        hrows[j, row, :] = hrows[j, row, :] * b
                pltpu.sync_copy(hrows.at[j], accum_sp.at[dst_v.at[j]],
                                add=True)
            return carry

        lax.fori_loop(0, STEPS, step_fn, 0)

        plsc.subcore_barrier()
        for k in range(NCH):
            r0 = out_r0 + k * BCH
            pltpu.sync_copy(accum_sp.at[pl.ds(r0, BCH)], vtmp)
            pltpu.sync_copy(vtmp, numer_hbm.at[c_id, c, pl.ds(r0, BCH)])
            if c + 1 < C:
                pltpu.sync_copy(zv, accum_sp.at[pl.ds(r0, BCH)])
        if c == 0:
            pltpu.sync_copy(denom_sp.at[pl.ds(out_r0, ROWS_TILE)], vtmpd)
            pltpu.sync_copy(vtmpd, denom_hbm.at[c_id, s_id, 0])


def _sc_edge_pass(C, E_prime, src2d, dst2d, h_list, as_t, ad_t,
                  zrows, zden):
    mesh = plsc.VectorSubcoreMesh(core_axis_name="c", subcore_axis_name="s")
    body = functools.partial(_sc_edge_body, C, E_prime)
    fn = pl.kernel(
        body,
        compiler_params=pltpu.CompilerParams(use_tc_tiling_on_sc=False),
        out_type=(
            jax.ShapeDtypeStruct((2, C, NP, 16), jnp.float32),
            jax.ShapeDtypeStruct((2, 16, 1, ROWS_TILE), jnp.float32),
            jax.ShapeDtypeStruct((R_EDGE, LANE), jnp.float32),
        ),
        mesh=mesh,
        scratch_types=[
            pltpu.VMEM_SHARED((NP, 16), jnp.float32),
            pltpu.VMEM_SHARED((NP,), jnp.float32),
            pltpu.VMEM((RPS, LANE), jnp.int32),
            pltpu.VMEM((RPS, LANE), jnp.int32),
            pltpu.VMEM((RPS, LANE), jnp.float32),
            pltpu.VMEM((RPS, LANE), jnp.float32),
            pltpu.VMEM((RPS, LANE), jnp.float32),
            pltpu.VMEM((RPS, LANE, 16), jnp.float32),
            pltpu.VMEM((136, 16), jnp.float32),
            pltpu.VMEM((136, 16), jnp.float32),
            pltpu.VMEM((ROWS_TILE,), jnp.float32),
            pltpu.SemaphoreType.DMA,
            pltpu.SemaphoreType.DMA,
        ],
    )
    return fn(src2d, dst2d, *h_list, as_t, ad_t, zrows, zden)


def kernel(x, edge_index, batch, W1, a1_src, a1_dst, b1, W2, a2_src, a2_dst,
           b2, Wl, bl):
    N = x.shape[0]
    E = edge_index.shape[1]
    E_prime = E + N
    pad = E_PAD - E_prime

    loops = jnp.arange(N, dtype=jnp.int32)
    zpad = jnp.zeros((pad,), jnp.int32)
    src2d = jnp.concatenate([edge_index[0].astype(jnp.int32), loops, zpad]
                            ).reshape(R_EDGE, LANE)
    dst2d = jnp.concatenate([edge_index[1].astype(jnp.int32), loops, zpad]
                            ).reshape(R_EDGE, LANE)
    zrows = jnp.zeros((136, 16), jnp.float32)
    zden = jnp.zeros((ROWS_TILE,), jnp.float32)

    h1, aa1 = _dense1(x, W1, a1_src, a1_dst)
    numer1, denom1, _ = _sc_edge_pass(
        1, E_prime, src2d, dst2d, [h1],
        aa1[:, 0].reshape(N), aa1[:, 1].reshape(N), zrows, zden)

    hc, aa2 = _combine1(numer1[:, 0], denom1.reshape(2, NP), b1, W2,
                        a2_src, a2_dst)
    numer2, denom2, _ = _sc_edge_pass(
        4, E_prime, src2d, dst2d, [hc[0], hc[1], hc[2], hc[3]],
        aa2[:, 0].reshape(NP), aa2[:, 1].reshape(NP), zrows, zden)

    out = _combine2(numer2, denom2.reshape(2, NP), b2, Wl, bl)
    return out[:N]
